# baseline (device time: 64742 ns/iter reference)
import jax
import jax.numpy as jnp
from jax import lax
from jax.experimental import pallas as pl
from jax.experimental.pallas import tpu as pltpu

N_DEV = 8


def kernel(x, Wq, Wo, K_ext, V_ext):
    B, Sq, D = x.shape
    Skv, Hkv, Dh = K_ext.shape[1:]
    d_local = Wq.shape[1]
    hq_local = d_local // Dh
    group = (hq_local * N_DEV) // Hkv
    kv_local = hq_local // group

    my = lax.axis_index("i")

    K_loc = lax.dynamic_slice_in_dim(K_ext, my * kv_local, kv_local, axis=2)
    V_loc = lax.dynamic_slice_in_dim(V_ext, my * kv_local, kv_local, axis=2)
    K_loc = K_loc.reshape(B, Skv, kv_local * Dh)
    V_loc = V_loc.reshape(B, Skv, kv_local * Dh)

    def body(x_ref, wq_ref, wo_ref, k_ref, v_ref, out_ref,
             comm_ref, send_sems, recv_sems):
        my_pos = lax.axis_index("i")
        right = lax.rem(my_pos + 1, N_DEV)
        left = lax.rem(my_pos + N_DEV - 1, N_DEV)

        barrier_sem = pltpu.get_barrier_semaphore()
        for nbr in (left, right):
            pl.semaphore_signal(
                barrier_sem, inc=1,
                device_id=(nbr,), device_id_type=pl.DeviceIdType.MESH,
            )
        pl.semaphore_wait(barrier_sem, 2)

        for b in range(B):
            q = jnp.dot(x_ref[b], wq_ref[...],
                        preferred_element_type=jnp.float32)
            o_heads = []
            for h in range(hq_local):
                g = h // group
                qh = q[:, h * Dh:(h + 1) * Dh]
                kh = k_ref[b, :, g * Dh:(g + 1) * Dh]
                vh = v_ref[b, :, g * Dh:(g + 1) * Dh]
                s = lax.dot_general(
                    qh, kh, (((1,), (1,)), ((), ())),
                    preferred_element_type=jnp.float32) * 0.125
                m = jnp.max(s, axis=1, keepdims=True)
                p = jnp.exp(s - m)
                l = jnp.sum(p, axis=1, keepdims=True)
                oh = jnp.dot(p, vh, preferred_element_type=jnp.float32) / l
                o_heads.append(oh)
            O = jnp.concatenate(o_heads, axis=1)
            comm_ref[0, b] = jnp.dot(O, wo_ref[...],
                                     preferred_element_type=jnp.float32)

        acc = comm_ref[0]
        for h in range(N_DEV - 1):
            rdma = pltpu.make_async_remote_copy(
                src_ref=comm_ref.at[h],
                dst_ref=comm_ref.at[h + 1],
                send_sem=send_sems.at[h],
                recv_sem=recv_sems.at[h],
                device_id=(right,),
                device_id_type=pl.DeviceIdType.MESH,
            )
            rdma.start()
            rdma.wait()
            acc = acc + comm_ref[h + 1]
        out_ref[...] = acc

    return pl.pallas_call(
        body,
        out_shape=jax.ShapeDtypeStruct((B, Sq, D), jnp.float32),
        in_specs=[pl.BlockSpec(memory_space=pltpu.VMEM)] * 5,
        out_specs=pl.BlockSpec(memory_space=pltpu.VMEM),
        scratch_shapes=[
            pltpu.VMEM((N_DEV, B, Sq, D), jnp.float32),
            pltpu.SemaphoreType.DMA((N_DEV - 1,)),
            pltpu.SemaphoreType.DMA((N_DEV - 1,)),
        ],
        compiler_params=pltpu.CompilerParams(collective_id=0),
    )(x, Wq, Wo, K_loc, V_loc)


# device time: 23858 ns/iter; 2.7136x vs baseline; 2.7136x over previous
import jax
import jax.numpy as jnp
from jax import lax
from jax.experimental import pallas as pl
from jax.experimental.pallas import tpu as pltpu

N_DEV = 8


def kernel(x, Wq, Wo, K_ext, V_ext):
    B, Sq, D = x.shape
    Skv, Hkv, Dh = K_ext.shape[1:]
    d_local = Wq.shape[1]
    hq_local = d_local // Dh
    group = (hq_local * N_DEV) // Hkv
    kv_local = hq_local // group
    rows = B * Sq
    crows = rows // N_DEV

    my = lax.axis_index("i")

    K_loc = lax.dynamic_slice_in_dim(K_ext, my * kv_local, kv_local, axis=2)
    V_loc = lax.dynamic_slice_in_dim(V_ext, my * kv_local, kv_local, axis=2)
    K_loc = K_loc.reshape(B, Skv, kv_local * Dh)
    V_loc = V_loc.reshape(B, Skv, kv_local * Dh)

    def body(x_ref, wq_ref, wo_ref, k_ref, v_ref, out_ref,
             p_ref, rs_ref, r_ref, ag_ref, send_sems, rs_sems, ag_sems):
        my_pos = lax.axis_index("i")

        barrier_sem = pltpu.get_barrier_semaphore()
        for p in range(N_DEV):
            @pl.when(p != my_pos)
            def _():
                pl.semaphore_signal(
                    barrier_sem, inc=1,
                    device_id=(p,), device_id_type=pl.DeviceIdType.MESH,
                )
        pl.semaphore_wait(barrier_sem, N_DEV - 1)

        for b in range(B):
            q = jnp.dot(x_ref[b], wq_ref[...],
                        preferred_element_type=jnp.float32)
            o_heads = []
            for h in range(hq_local):
                g = h // group
                qh = q[:, h * Dh:(h + 1) * Dh]
                kh = k_ref[b, :, g * Dh:(g + 1) * Dh]
                vh = v_ref[b, :, g * Dh:(g + 1) * Dh]
                s = lax.dot_general(
                    qh, kh, (((1,), (1,)), ((), ())),
                    preferred_element_type=jnp.float32) * 0.125
                m = jnp.max(s, axis=1, keepdims=True)
                p = jnp.exp(s - m)
                l = jnp.sum(p, axis=1, keepdims=True)
                oh = jnp.dot(p, vh, preferred_element_type=jnp.float32) / l
                o_heads.append(oh)
            O = jnp.concatenate(o_heads, axis=1)
            p_ref[b * Sq:(b + 1) * Sq, :] = jnp.dot(
                O, wo_ref[...], preferred_element_type=jnp.float32)

        rs_sends = []
        for j in range(N_DEV):
            rdma = pltpu.make_async_remote_copy(
                src_ref=p_ref.at[pl.ds(j * crows, crows), :],
                dst_ref=rs_ref.at[my_pos],
                send_sem=send_sems.at[j],
                recv_sem=rs_sems.at[my_pos],
                device_id=(j,),
                device_id_type=pl.DeviceIdType.MESH,
            )
            rs_sends.append(rdma)

            @pl.when(j != my_pos)
            def _():
                rdma.start()

        acc = p_ref[pl.ds(my_pos * crows, crows), :]
        for s in range(N_DEV):
            recv = pltpu.make_async_remote_copy(
                src_ref=p_ref.at[pl.ds(0, crows), :],
                dst_ref=rs_ref.at[s],
                send_sem=send_sems.at[s],
                recv_sem=rs_sems.at[s],
                device_id=(s,),
                device_id_type=pl.DeviceIdType.MESH,
            )

            @pl.when(s != my_pos)
            def _():
                recv.wait_recv()
            acc = acc + jnp.where(s == my_pos,
                                  jnp.zeros((crows, D), jnp.float32),
                                  rs_ref[s])
        r_ref[...] = acc

        for j in range(N_DEV):
            @pl.when(j != my_pos)
            def _():
                rs_sends[j].wait_send()

        ag_sends = []
        for j in range(N_DEV):
            rdma = pltpu.make_async_remote_copy(
                src_ref=r_ref,
                dst_ref=ag_ref.at[my_pos],
                send_sem=send_sems.at[j],
                recv_sem=ag_sems.at[my_pos],
                device_id=(j,),
                device_id_type=pl.DeviceIdType.MESH,
            )
            ag_sends.append(rdma)

            @pl.when(j != my_pos)
            def _():
                rdma.start()

        for o in range(N_DEV):
            recv = pltpu.make_async_remote_copy(
                src_ref=r_ref,
                dst_ref=ag_ref.at[o],
                send_sem=send_sems.at[o],
                recv_sem=ag_sems.at[o],
                device_id=(o,),
                device_id_type=pl.DeviceIdType.MESH,
            )

            @pl.when(o != my_pos)
            def _():
                recv.wait_recv()
            chunk = jnp.where(o == my_pos, r_ref[...], ag_ref[o])
            b = (o * crows) // Sq
            r0 = (o * crows) % Sq
            out_ref[b, r0:r0 + crows, :] = chunk

        for j in range(N_DEV):
            @pl.when(j != my_pos)
            def _():
                ag_sends[j].wait_send()

    return pl.pallas_call(
        body,
        out_shape=jax.ShapeDtypeStruct((B, Sq, D), jnp.float32),
        in_specs=[pl.BlockSpec(memory_space=pltpu.VMEM)] * 5,
        out_specs=pl.BlockSpec(memory_space=pltpu.VMEM),
        scratch_shapes=[
            pltpu.VMEM((rows, D), jnp.float32),
            pltpu.VMEM((N_DEV, crows, D), jnp.float32),
            pltpu.VMEM((crows, D), jnp.float32),
            pltpu.VMEM((N_DEV, crows, D), jnp.float32),
            pltpu.SemaphoreType.DMA((N_DEV,)),
            pltpu.SemaphoreType.DMA((N_DEV,)),
            pltpu.SemaphoreType.DMA((N_DEV,)),
        ],
        compiler_params=pltpu.CompilerParams(collective_id=0),
    )(x, Wq, Wo, K_loc, V_loc)


# device time: 19163 ns/iter; 3.3785x vs baseline; 1.2450x over previous
import jax
import jax.numpy as jnp
from jax import lax
from jax.experimental import pallas as pl
from jax.experimental.pallas import tpu as pltpu

N_DEV = 8


def kernel(x, Wq, Wo, K_ext, V_ext):
    B, Sq, D = x.shape
    Skv, Hkv, Dh = K_ext.shape[1:]
    d_local = Wq.shape[1]
    hq_local = d_local // Dh
    group = (hq_local * N_DEV) // Hkv
    kv_local = hq_local // group
    rows = B * Sq
    crows = rows // N_DEV

    my = lax.axis_index("i")

    K_loc = lax.dynamic_slice_in_dim(K_ext, my * kv_local, kv_local, axis=2)
    V_loc = lax.dynamic_slice_in_dim(V_ext, my * kv_local, kv_local, axis=2)
    K_loc = K_loc.reshape(B, Skv, kv_local * Dh)
    V_loc = V_loc.reshape(B, Skv, kv_local * Dh)

    def body(x_ref, wq_ref, wo_ref, k_ref, v_ref, out_ref,
             p_ref, rs_ref, r_ref, ag_ref, send_sems, rs_sems, ag_sems):
        my_pos = lax.axis_index("i")

        barrier_sem = pltpu.get_barrier_semaphore()
        for p in range(N_DEV):
            @pl.when(p != my_pos)
            def _():
                pl.semaphore_signal(
                    barrier_sem, inc=1,
                    device_id=(p,), device_id_type=pl.DeviceIdType.MESH,
                )
        pl.semaphore_wait(barrier_sem, N_DEV - 1)

        q = jnp.dot(x_ref[...].reshape(rows, D), wq_ref[...],
                    preferred_element_type=jnp.float32)
        o_blocks = [[None] * hq_local for _ in range(B)]
        for b in range(B):
            for g in range(kv_local):
                qg = jnp.concatenate(
                    [q[b * Sq:(b + 1) * Sq,
                       (g * group + t) * Dh:(g * group + t + 1) * Dh]
                     for t in range(group)], axis=0)
                kh = k_ref[b, :, g * Dh:(g + 1) * Dh]
                vh = v_ref[b, :, g * Dh:(g + 1) * Dh]
                s = lax.dot_general(
                    qg, kh, (((1,), (1,)), ((), ())),
                    preferred_element_type=jnp.float32) * 0.125
                m = jnp.max(s, axis=1, keepdims=True)
                p = jnp.exp(s - m)
                l = jnp.sum(p, axis=1, keepdims=True)
                og = jnp.dot(p, vh, preferred_element_type=jnp.float32) / l
                for t in range(group):
                    o_blocks[b][g * group + t] = og[t * Sq:(t + 1) * Sq]
        O = jnp.concatenate(
            [jnp.concatenate(o_blocks[b], axis=1) for b in range(B)],
            axis=0)
        p_ref[...] = jnp.dot(O, wo_ref[...],
                             preferred_element_type=jnp.float32)

        rs_sends = []
        for j in range(N_DEV):
            rdma = pltpu.make_async_remote_copy(
                src_ref=p_ref.at[pl.ds(j * crows, crows), :],
                dst_ref=rs_ref.at[my_pos],
                send_sem=send_sems.at[j],
                recv_sem=rs_sems.at[my_pos],
                device_id=(j,),
                device_id_type=pl.DeviceIdType.MESH,
            )
            rs_sends.append(rdma)

            @pl.when(j != my_pos)
            def _():
                rdma.start()

        acc = p_ref[pl.ds(my_pos * crows, crows), :]
        for s in range(N_DEV):
            recv = pltpu.make_async_remote_copy(
                src_ref=p_ref.at[pl.ds(0, crows), :],
                dst_ref=rs_ref.at[s],
                send_sem=send_sems.at[s],
                recv_sem=rs_sems.at[s],
                device_id=(s,),
                device_id_type=pl.DeviceIdType.MESH,
            )

            @pl.when(s != my_pos)
            def _():
                recv.wait_recv()
            acc = acc + jnp.where(s == my_pos,
                                  jnp.zeros((crows, D), jnp.float32),
                                  rs_ref[s])
        r_ref[...] = acc

        for j in range(N_DEV):
            @pl.when(j != my_pos)
            def _():
                rs_sends[j].wait_send()

        ag_sends = []
        for j in range(N_DEV):
            rdma = pltpu.make_async_remote_copy(
                src_ref=r_ref,
                dst_ref=ag_ref.at[my_pos],
                send_sem=send_sems.at[j],
                recv_sem=ag_sems.at[my_pos],
                device_id=(j,),
                device_id_type=pl.DeviceIdType.MESH,
            )
            ag_sends.append(rdma)

            @pl.when(j != my_pos)
            def _():
                rdma.start()

        for o in range(N_DEV):
            recv = pltpu.make_async_remote_copy(
                src_ref=r_ref,
                dst_ref=ag_ref.at[o],
                send_sem=send_sems.at[o],
                recv_sem=ag_sems.at[o],
                device_id=(o,),
                device_id_type=pl.DeviceIdType.MESH,
            )

            @pl.when(o != my_pos)
            def _():
                recv.wait_recv()
            chunk = jnp.where(o == my_pos, r_ref[...], ag_ref[o])
            b = (o * crows) // Sq
            r0 = (o * crows) % Sq
            out_ref[b, r0:r0 + crows, :] = chunk

        for j in range(N_DEV):
            @pl.when(j != my_pos)
            def _():
                ag_sends[j].wait_send()

    return pl.pallas_call(
        body,
        out_shape=jax.ShapeDtypeStruct((B, Sq, D), jnp.float32),
        in_specs=[pl.BlockSpec(memory_space=pltpu.VMEM)] * 5,
        out_specs=pl.BlockSpec(memory_space=pltpu.VMEM),
        scratch_shapes=[
            pltpu.VMEM((rows, D), jnp.float32),
            pltpu.VMEM((N_DEV, crows, D), jnp.float32),
            pltpu.VMEM((crows, D), jnp.float32),
            pltpu.VMEM((N_DEV, crows, D), jnp.float32),
            pltpu.SemaphoreType.DMA((N_DEV,)),
            pltpu.SemaphoreType.DMA((N_DEV,)),
            pltpu.SemaphoreType.DMA((N_DEV,)),
        ],
        compiler_params=pltpu.CompilerParams(collective_id=0),
    )(x, Wq, Wo, K_loc, V_loc)
